# baseline (speedup 1.0000x reference)
import jax
import jax.numpy as jnp
from jax.experimental import pallas as pl

_B, _V = 8, 1000000
_NC, _W = 40, 25000
_NSTEP = _NC // 8
_EPS = 1e-10


def _rotl(x, d):
    return (x << jnp.uint32(d)) | (x >> jnp.uint32(32 - d))


def _threefry_bits(idx):
    ks0 = jnp.uint32(0)
    ks1 = jnp.uint32(1)
    ks2 = jnp.uint32(0x1BD11BDA) ^ ks0 ^ ks1
    ks = (ks0, ks1, ks2)
    rots = ((13, 15, 26, 6), (17, 29, 16, 24))
    x0 = jnp.zeros_like(idx) + ks0
    x1 = idx + ks1
    for i in range(5):
        for r in rots[i % 2]:
            x0 = x0 + x1
            x1 = _rotl(x1, r) ^ x0
        x0 = x0 + ks[(i + 1) % 3]
        x1 = x1 + ks[(i + 2) % 3] + jnp.uint32(i + 1)
    return x0 ^ x1


def _row_kernel(a_ref, y_ref, ss_ref):
    row = pl.program_id(0)
    eps = jnp.float32(_EPS)

    rw = jax.lax.broadcasted_iota(jnp.uint32, (8, _W), 0) * jnp.uint32(_W) + \
         jax.lax.broadcasted_iota(jnp.uint32, (8, _W), 1)
    row_base = (row * _V).astype(jnp.uint32)

    mg = jnp.float32(-jnp.inf)
    ma = jnp.float32(-jnp.inf)
    for k in range(_NSTEP):
        a = a_ref[0, pl.ds(k * 8, 8), :]
        idx = rw + (row_base + jnp.uint32(k * 8 * _W))
        bits = _threefry_bits(idx)
        u = jax.lax.bitcast_convert_type(
            (bits >> jnp.uint32(9)) | jnp.uint32(0x3F800000), jnp.float32
        ) - jnp.float32(1.0)
        g = a - jnp.log(eps - jnp.log(u + eps))
        y_ref[0, pl.ds(k * 8, 8), :] = g
        mg = jnp.maximum(mg, jnp.max(g))
        ma = jnp.maximum(ma, jnp.max(a))

    sg = jnp.float32(0.0)
    sa = jnp.float32(0.0)
    for k in range(_NSTEP):
        e = jnp.exp(y_ref[0, pl.ds(k * 8, 8), :] - mg)
        y_ref[0, pl.ds(k * 8, 8), :] = e
        sg = sg + jnp.sum(e)
        e2 = jnp.exp(a_ref[0, pl.ds(k * 8, 8), :] - ma)
        ss_ref[0, pl.ds(k * 8, 8), :] = e2
        sa = sa + jnp.sum(e2)

    rg = jnp.float32(1.0) / sg
    ra = jnp.float32(1.0) / sa
    for k in range(_NSTEP):
        y_ref[0, pl.ds(k * 8, 8), :] = jnp.maximum(
            y_ref[0, pl.ds(k * 8, 8), :] * rg, eps)
        ss_ref[0, pl.ds(k * 8, 8), :] = ss_ref[0, pl.ds(k * 8, 8), :] * ra


def kernel(alpha):
    a3 = alpha.reshape(_B, _NC, _W)
    y, ss = pl.pallas_call(
        _row_kernel,
        grid=(_B,),
        in_specs=[pl.BlockSpec((1, _NC, _W), lambda i: (i, 0, 0))],
        out_specs=[
            pl.BlockSpec((1, _NC, _W), lambda i: (i, 0, 0)),
            pl.BlockSpec((1, _NC, _W), lambda i: (i, 0, 0)),
        ],
        out_shape=[
            jax.ShapeDtypeStruct((_B, _NC, _W), jnp.float32),
            jax.ShapeDtypeStruct((_B, _NC, _W), jnp.float32),
        ],
    )(a3)
    y = y.reshape(_B, _V)
    ss = ss.reshape(_B, _V)
    return (y, y, ss)
